# Initial kernel scaffold; baseline (speedup 1.0000x reference)
import jax
import jax.numpy as jnp
from jax.experimental import pallas as pl


def kernel(xyz, edge_index, edge_type_idx, edge_rest_lengths, edge_type_emb, W1, b1, W2, b2, non_edge_bias):
    raise NotImplementedError("write your pallas kernel here")


# speedup vs baseline: 17.8320x; 17.8320x over previous
import functools

import jax
import jax.numpy as jnp
from jax import lax
from jax.experimental import pallas as pl
from jax.experimental.pallas import tpu as pltpu
from jax.experimental.pallas import tpu_sc as plsc

B, N, E = 4, 2048, 32768
BE = B * E
NT = 32
EPT = E // NT
NF = 16

ROWS_PER_TILE = 256
TILES_PER_BATCH = N // ROWS_PER_TILE
BLK_ROWS = 16
NBLK = ROWS_PER_TILE // BLK_ROWS
BLK_W = BLK_ROWS * N
LIST_CAP = 10240
LIST_DUMP = LIST_CAP + 16

_MESH = dict(core_axis_name="c", subcore_axis_name="s", num_cores=2,
             num_subcores=16)


def _widx():
    return lax.axis_index("s") * 2 + lax.axis_index("c")


def _rsqrt_pos(x):
    ib = plsc.bitcast(x, jnp.int32)
    y = plsc.bitcast(jnp.int32(0x5F3759DF) - (ib >> 1), jnp.float32)
    for _ in range(3):
        y = y * (1.5 - 0.5 * x * y * y)
    return y


def _geom_body(xyz_hbm, i_hbm, j_hbm, rest_hbm, type_hbm, emb_hbm, feat_hbm,
               xyz_v, iv_v, jv_v, rest_v, type_v, emb_v, fb_v, sem):
    w = _widx()
    e0 = w * EPT
    pltpu.sync_copy(xyz_hbm, xyz_v)
    pltpu.sync_copy(i_hbm.at[pl.ds(e0, EPT)], iv_v)
    pltpu.sync_copy(j_hbm.at[pl.ds(e0, EPT)], jv_v)
    pltpu.sync_copy(rest_hbm.at[pl.ds(e0, EPT)], rest_v)
    pltpu.sync_copy(type_hbm.at[pl.ds(e0, EPT)], type_v)
    pltpu.sync_copy(emb_hbm, emb_v)

    zero16 = jnp.zeros((16,), jnp.float32)

    def chunk(c, carry):
        sl = pl.ds(c * 16, 16)
        iv = iv_v[sl]
        jv = jv_v[sl]
        tv = type_v[sl] * 8
        restv = rest_v[sl]
        inv = 1.0 / (restv + 1e-9)
        iv3 = iv * 3
        jv3 = jv * 3
        for d in range(8):
            sd = plsc.load_gather(emb_v, [tv + d])
            for b in range(B):
                fb_v[b, 2 + d, sl] = sd
        for b in range(B):
            off = b * (N * 3)
            diffs = []
            for d in range(3):
                pi = plsc.load_gather(xyz_v, [iv3 + (off + d)])
                pj = plsc.load_gather(xyz_v, [jv3 + (off + d)])
                diffs.append(pi - pj)
            dist2 = diffs[0] * diffs[0] + diffs[1] * diffs[1] + diffs[2] * diffs[2]
            x = dist2 + 1e-9
            dist = x * _rsqrt_pos(x)
            delta = (dist - restv) * inv
            fb_v[b, 0, sl] = dist2
            fb_v[b, 1, sl] = delta
            for f in range(10, NF):
                fb_v[b, f, sl] = zero16
        return carry

    lax.fori_loop(0, EPT // 16, chunk, 0)

    cps = []
    for b in range(B):
        col0 = b * E + e0
        cps.append(pltpu.async_copy(fb_v.at[b], feat_hbm.at[:, pl.ds(col0, EPT)],
                                    sem))
    for cp in cps:
        cp.wait()


def _geom(xyz, i, j, rest, tidx, emb):
    mesh = plsc.VectorSubcoreMesh(**_MESH)
    return pl.kernel(
        _geom_body,
        out_type=jax.ShapeDtypeStruct((NF, BE), jnp.float32),
        mesh=mesh,
        compiler_params=pltpu.CompilerParams(needs_layout_passes=False),
        scratch_types=[
            pltpu.VMEM((B * N * 3,), jnp.float32),
            pltpu.VMEM((EPT,), jnp.int32),
            pltpu.VMEM((EPT,), jnp.int32),
            pltpu.VMEM((EPT,), jnp.float32),
            pltpu.VMEM((EPT,), jnp.int32),
            pltpu.VMEM((16 * 8,), jnp.float32),
            pltpu.VMEM((B, NF, EPT), jnp.float32),
            pltpu.SemaphoreType.DMA,
        ],
    )(xyz, i, j, rest, tidx, emb)


def _mlp_body(w1t_ref, b1_ref, w2t_ref, b2_ref, feat_ref, out_ref):
    f = feat_ref[...]
    mask = lax.broadcasted_iota(jnp.int32, (NF, 1), 0) < 10
    f = jnp.where(mask, f, 0.0)
    h = jnp.dot(w1t_ref[...], f, preferred_element_type=jnp.float32)
    h = h + b1_ref[...]
    h = h * (1.0 / (1.0 + jnp.exp(-h)))
    o = jnp.dot(w2t_ref[...], h, preferred_element_type=jnp.float32)
    out_ref[...] = o + b2_ref[0]


def _mlp(w1t, b1c, w2t, b2, feat):
    blk = 1024
    return pl.pallas_call(
        _mlp_body,
        grid=(BE // blk,),
        in_specs=[
            pl.BlockSpec((32, NF), lambda g: (0, 0)),
            pl.BlockSpec((32, 1), lambda g: (0, 0)),
            pl.BlockSpec((1, 32), lambda g: (0, 0)),
            pl.BlockSpec(memory_space=pltpu.SMEM),
            pl.BlockSpec((NF, blk), lambda g: (0, g)),
        ],
        out_specs=pl.BlockSpec((1, blk), lambda g: (0, g)),
        out_shape=jax.ShapeDtypeStruct((1, BE), jnp.float32),
    )(w1t, b1c, w2t, b2, feat)


def _scatter_body(i_hbm, j_hbm, eb_hbm, nb_hbm, out_hbm,
                  iv_st, jv_st, listI, listE, vals, bufA, bufB, nb_v,
                  semA, semB):
    w = _widx()
    b = w // TILES_PER_BATCH
    r0 = (w % TILES_PER_BATCH) * ROWS_PER_TILE
    pltpu.sync_copy(nb_hbm, nb_v)
    nbvec = nb_v[...]
    iota = jnp.arange(16, dtype=jnp.int32)

    cnt = jnp.int32(0)
    for p in range(2):
        for s in range(E // 2048):
            pltpu.sync_copy(i_hbm.at[pl.ds(s * 2048, 2048)], iv_st)
            pltpu.sync_copy(j_hbm.at[pl.ds(s * 2048, 2048)], jv_st)

            def sbody(c, cnt, p=p, s=s):
                sl = pl.ds(c * 16, 16)
                iv = iv_st[sl]
                jv = jv_st[sl]
                rows = iv if p == 0 else jv
                cols = jv if p == 0 else iv
                m = (rows >= r0) & (rows < r0 + ROWS_PER_TILE)
                mi = m.astype(jnp.int32)
                cs = plsc.cumsum(mi)
                rank = cs - mi
                ok = m & (cnt < LIST_CAP)
                dest = jnp.where(ok, cnt + rank, LIST_DUMP)
                lidx = (rows - r0) * N + cols
                eid = (s * 2048 + c * 16) + iota
                plsc.store_scatter(listI, [dest], lidx)
                plsc.store_scatter(listE, [dest], eid)
                return jnp.minimum(cnt + jnp.max(cs), LIST_CAP)

            cnt = lax.fori_loop(0, 128, sbody, cnt)

    listE[pl.ds(cnt, 16)] = jnp.zeros((16,), jnp.int32)
    listI[pl.ds(cnt, 16)] = jnp.zeros((16,), jnp.int32)
    nch = (cnt + 15) // 16

    pltpu.sync_copy(eb_hbm.at[pl.ds(b * E, E)], bufB.at[pl.ds(0, E)])

    def vbody(n, carry):
        base = n * 16
        eids = listE[pl.ds(base, 16)]
        v = plsc.load_gather(bufB, [eids])
        vals[pl.ds(base, 16)] = v
        return carry

    lax.fori_loop(0, nch, vbody, 0)

    def fbody(t, carry):
        bufA[pl.ds(t * 16, 16)] = nbvec
        bufB[pl.ds(t * 16, 16)] = nbvec
        return carry

    lax.fori_loop(0, BLK_W // 16, fbody, 0)

    cps = [None, None]
    for k in range(NBLK):
        buf = bufA if k % 2 == 0 else bufB
        sem = semA if k % 2 == 0 else semB
        if k >= 2:
            cps[k % 2].wait()
            prev = k - 2

            def rbody(n, carry, buf=buf, prev=prev):
                base = n * 16
                li = listI[pl.ds(base, 16)]
                m = ((base + iota) < cnt) & ((li >> 15) == prev)
                dest = jnp.where(m, li & (BLK_W - 1), BLK_W)
                plsc.store_scatter(buf, [dest], nbvec)
                return carry

            lax.fori_loop(0, nch, rbody, 0)

        def abody(n, carry, buf=buf, k=k):
            base = n * 16
            li = listI[pl.ds(base, 16)]
            vv = vals[pl.ds(base, 16)]
            m = ((base + iota) < cnt) & ((li >> 15) == k)
            dest = jnp.where(m, li & (BLK_W - 1), BLK_W)
            plsc.store_scatter(buf, [dest], vv)
            return carry

        lax.fori_loop(0, nch, abody, 0)
        obase = (b * N + r0 + k * BLK_ROWS) * N
        cps[k % 2] = pltpu.async_copy(buf.at[pl.ds(0, BLK_W)],
                                      out_hbm.at[pl.ds(obase, BLK_W)], sem)
    cps[0].wait()
    cps[1].wait()


def _scatter(i, j, ebf, nbv):
    mesh = plsc.VectorSubcoreMesh(**_MESH)
    return pl.kernel(
        _scatter_body,
        out_type=jax.ShapeDtypeStruct((B * N * N,), jnp.float32),
        mesh=mesh,
        compiler_params=pltpu.CompilerParams(needs_layout_passes=False),
        scratch_types=[
            pltpu.VMEM((2048,), jnp.int32),
            pltpu.VMEM((2048,), jnp.int32),
            pltpu.VMEM((LIST_CAP + 32,), jnp.int32),
            pltpu.VMEM((LIST_CAP + 32,), jnp.int32),
            pltpu.VMEM((LIST_CAP + 32,), jnp.float32),
            pltpu.VMEM((BLK_W + 16,), jnp.float32),
            pltpu.VMEM((BLK_W + 16,), jnp.float32),
            pltpu.VMEM((16,), jnp.float32),
            pltpu.SemaphoreType.DMA,
            pltpu.SemaphoreType.DMA,
        ],
    )(i, j, ebf, nbv)


def kernel(xyz, edge_index, edge_type_idx, edge_rest_lengths, edge_type_emb,
           W1, b1, W2, b2, non_edge_bias):
    f32 = jnp.float32
    i = edge_index[0].astype(jnp.int32)
    j = edge_index[1].astype(jnp.int32)
    tidx = edge_type_idx.astype(jnp.int32)
    rest = edge_rest_lengths.astype(f32)

    feat = _geom(xyz.astype(f32).reshape(-1), i, j, rest, tidx,
                 edge_type_emb.astype(f32).reshape(-1))

    w1t = jnp.zeros((32, NF), f32).at[:, :10].set(W1.astype(f32).T)
    eb = _mlp(w1t, b1.astype(f32).reshape(32, 1), W2.astype(f32).T.reshape(1, 32),
              b2.astype(f32).reshape(1), feat)

    nbv = jnp.broadcast_to(non_edge_bias.astype(f32).reshape(1), (16,))
    flat = _scatter(i, j, eb.reshape(BE), nbv)
    return flat.reshape(B, 1, N, N)


# speedup vs baseline: 19.6316x; 1.1009x over previous
import functools

import jax
import jax.numpy as jnp
from jax import lax
from jax.experimental import pallas as pl
from jax.experimental.pallas import tpu as pltpu
from jax.experimental.pallas import tpu_sc as plsc

B, N, E = 4, 2048, 32768
BE = B * E
NT = 32
EPT = E // NT
NF = 16

ROWS_PER_TILE = 256
TILES_PER_BATCH = N // ROWS_PER_TILE
BLK_ROWS = 16
NBLK = ROWS_PER_TILE // BLK_ROWS
BLK_W = BLK_ROWS * N
LIST_CAP = 10240
LIST_DUMP = LIST_CAP + 16

_MESH = dict(core_axis_name="c", subcore_axis_name="s", num_cores=2,
             num_subcores=16)


def _widx():
    return lax.axis_index("s") * 2 + lax.axis_index("c")


def _rsqrt_pos(x):
    ib = plsc.bitcast(x, jnp.int32)
    y = plsc.bitcast(jnp.int32(0x5F3759DF) - (ib >> 1), jnp.float32)
    for _ in range(3):
        y = y * (1.5 - 0.5 * x * y * y)
    return y


def _geom_body(xyz_hbm, i_hbm, j_hbm, rest_hbm, type_hbm, emb_hbm, feat_hbm,
               xyz_v, iv_v, jv_v, rest_v, type_v, emb_v, fb_v, sem):
    w = _widx()
    e0 = w * EPT
    pltpu.sync_copy(xyz_hbm, xyz_v)
    pltpu.sync_copy(i_hbm.at[pl.ds(e0, EPT)], iv_v)
    pltpu.sync_copy(j_hbm.at[pl.ds(e0, EPT)], jv_v)
    pltpu.sync_copy(rest_hbm.at[pl.ds(e0, EPT)], rest_v)
    pltpu.sync_copy(type_hbm.at[pl.ds(e0, EPT)], type_v)
    pltpu.sync_copy(emb_hbm, emb_v)

    zero16 = jnp.zeros((16,), jnp.float32)

    def chunk(c, carry):
        sl = pl.ds(c * 16, 16)
        iv = iv_v[sl]
        jv = jv_v[sl]
        tv = type_v[sl] * 8
        restv = rest_v[sl]
        inv = 1.0 / (restv + 1e-9)
        iv3 = iv * 3
        jv3 = jv * 3
        for d in range(8):
            sd = plsc.load_gather(emb_v, [tv + d])
            for b in range(B):
                fb_v[b, 2 + d, sl] = sd
        for b in range(B):
            off = b * (N * 3)
            diffs = []
            for d in range(3):
                pi = plsc.load_gather(xyz_v, [iv3 + (off + d)])
                pj = plsc.load_gather(xyz_v, [jv3 + (off + d)])
                diffs.append(pi - pj)
            dist2 = diffs[0] * diffs[0] + diffs[1] * diffs[1] + diffs[2] * diffs[2]
            x = dist2 + 1e-9
            dist = x * _rsqrt_pos(x)
            delta = (dist - restv) * inv
            fb_v[b, 0, sl] = dist2
            fb_v[b, 1, sl] = delta
            for f in range(10, NF):
                fb_v[b, f, sl] = zero16
        return carry

    lax.fori_loop(0, EPT // 16, chunk, 0)

    cps = []
    for b in range(B):
        col0 = b * E + e0
        cps.append(pltpu.async_copy(fb_v.at[b], feat_hbm.at[:, pl.ds(col0, EPT)],
                                    sem))
    for cp in cps:
        cp.wait()


def _geom(xyz, i, j, rest, tidx, emb):
    mesh = plsc.VectorSubcoreMesh(**_MESH)
    return pl.kernel(
        _geom_body,
        out_type=jax.ShapeDtypeStruct((NF, BE), jnp.float32),
        mesh=mesh,
        compiler_params=pltpu.CompilerParams(needs_layout_passes=False),
        scratch_types=[
            pltpu.VMEM((B * N * 3,), jnp.float32),
            pltpu.VMEM((EPT,), jnp.int32),
            pltpu.VMEM((EPT,), jnp.int32),
            pltpu.VMEM((EPT,), jnp.float32),
            pltpu.VMEM((EPT,), jnp.int32),
            pltpu.VMEM((16 * 8,), jnp.float32),
            pltpu.VMEM((B, NF, EPT), jnp.float32),
            pltpu.SemaphoreType.DMA,
        ],
    )(xyz, i, j, rest, tidx, emb)


def _mlp_body(w1t_ref, b1_ref, w2t_ref, b2_ref, feat_ref, out_ref):
    f = feat_ref[...]
    mask = lax.broadcasted_iota(jnp.int32, (NF, 1), 0) < 10
    f = jnp.where(mask, f, 0.0)
    h = jnp.dot(w1t_ref[...], f, preferred_element_type=jnp.float32)
    h = h + b1_ref[...]
    h = h * (1.0 / (1.0 + jnp.exp(-h)))
    o = jnp.dot(w2t_ref[...], h, preferred_element_type=jnp.float32)
    out_ref[...] = o + b2_ref[0]


def _mlp(w1t, b1c, w2t, b2, feat):
    blk = 8192
    return pl.pallas_call(
        _mlp_body,
        grid=(BE // blk,),
        in_specs=[
            pl.BlockSpec((32, NF), lambda g: (0, 0)),
            pl.BlockSpec((32, 1), lambda g: (0, 0)),
            pl.BlockSpec((1, 32), lambda g: (0, 0)),
            pl.BlockSpec(memory_space=pltpu.SMEM),
            pl.BlockSpec((NF, blk), lambda g: (0, g)),
        ],
        out_specs=pl.BlockSpec((1, blk), lambda g: (0, g)),
        out_shape=jax.ShapeDtypeStruct((1, BE), jnp.float32),
    )(w1t, b1c, w2t, b2, feat)


def _scatter_body(i_hbm, j_hbm, eb_hbm, nb_hbm, out_hbm,
                  iv_st, jv_st, listI, listE, vals, bufA, bufB, nb_v, cnt_v,
                  semA, semB, sem_st):
    w = _widx()
    b = w // TILES_PER_BATCH
    r0 = (w % TILES_PER_BATCH) * ROWS_PER_TILE
    pltpu.sync_copy(nb_hbm, nb_v)
    nbvec = nb_v[...]
    iota = jnp.arange(16, dtype=jnp.int32)

    segs = [(p, s) for p in range(2) for s in range(E // 2048)]

    def _stage_in(t):
        slot = t % 2
        s = segs[t][1]
        return (pltpu.async_copy(i_hbm.at[pl.ds(s * 2048, 2048)],
                                 iv_st.at[slot], sem_st),
                pltpu.async_copy(j_hbm.at[pl.ds(s * 2048, 2048)],
                                 jv_st.at[slot], sem_st))

    pend = _stage_in(0)
    cnt = jnp.zeros((16,), jnp.int32)
    for t, (p, s) in enumerate(segs):
        for cp in pend:
            cp.wait()
        if t + 1 < len(segs):
            nxt = _stage_in(t + 1)
        slot = t % 2

        def sbody(c, cnt, p=p, s=s, slot=slot):
            sl = pl.ds(c * 16, 16)
            iv = iv_st[slot, sl]
            jv = jv_st[slot, sl]
            rows = iv if p == 0 else jv
            cols = jv if p == 0 else iv
            m = (rows >= r0) & (rows < r0 + ROWS_PER_TILE)
            mi = m.astype(jnp.int32)
            cs = plsc.cumsum(mi)
            rank = cs - mi
            ok = m & (cnt < LIST_CAP)
            dest = jnp.where(ok, cnt + rank, LIST_DUMP)
            lidx = (rows - r0) * N + cols
            eid = (s * 2048 + c * 16) + iota
            plsc.store_scatter(listI, [dest], lidx)
            plsc.store_scatter(listE, [dest], eid)
            pc = plsc.all_reduce_population_count(m)
            return jnp.minimum(cnt + pc, LIST_CAP)

        cnt = lax.fori_loop(0, 128, sbody, cnt)
        if t + 1 < len(segs):
            pend = nxt

    plsc.store_scatter(listI, [cnt + iota], jnp.full((16,), NBLK * BLK_W, jnp.int32))
    plsc.store_scatter(listE, [cnt + iota], jnp.zeros((16,), jnp.int32))
    cnt_v[...] = cnt
    cnt_s = jnp.max(cnt_v[...])
    nch = (cnt_s + 15) // 16

    pltpu.sync_copy(eb_hbm.at[pl.ds(b * E, E)], bufB.at[pl.ds(0, E)])

    def vbody(n, carry):
        base = n * 16
        eids = listE[pl.ds(base, 16)]
        v = plsc.load_gather(bufB, [eids])
        vals[pl.ds(base, 16)] = v
        return carry

    lax.fori_loop(0, nch, vbody, 0)

    def _fill(buf):
        def fbody(tt, carry):
            buf[pl.ds(tt * 16, 16)] = nbvec
            return carry
        lax.fori_loop(0, BLK_W // 16, fbody, 0)

    _fill(bufA)
    _fill(bufB)

    cps = [None, None]
    for k in range(NBLK):
        buf = bufA if k % 2 == 0 else bufB
        sem = semA if k % 2 == 0 else semB
        if k >= 2:
            cps[k % 2].wait()
            _fill(buf)

        def abody(n, carry, buf=buf, k=k):
            base = n * 16
            li = listI[pl.ds(base, 16)]
            vv = vals[pl.ds(base, 16)]
            m = (li >> 15) == k
            dest = jnp.where(m, li & (BLK_W - 1), BLK_W)
            plsc.store_scatter(buf, [dest], vv)
            return carry

        lax.fori_loop(0, nch, abody, 0)
        obase = (b * N + r0 + k * BLK_ROWS) * N
        cps[k % 2] = pltpu.async_copy(buf.at[pl.ds(0, BLK_W)],
                                      out_hbm.at[pl.ds(obase, BLK_W)], sem)
    cps[0].wait()
    cps[1].wait()


def _scatter(i, j, ebf, nbv):
    mesh = plsc.VectorSubcoreMesh(**_MESH)
    return pl.kernel(
        _scatter_body,
        out_type=jax.ShapeDtypeStruct((B * N * N,), jnp.float32),
        mesh=mesh,
        compiler_params=pltpu.CompilerParams(needs_layout_passes=False),
        scratch_types=[
            pltpu.VMEM((2, 2048), jnp.int32),
            pltpu.VMEM((2, 2048), jnp.int32),
            pltpu.VMEM((LIST_CAP + 32,), jnp.int32),
            pltpu.VMEM((LIST_CAP + 32,), jnp.int32),
            pltpu.VMEM((LIST_CAP + 32,), jnp.float32),
            pltpu.VMEM((BLK_W + 16,), jnp.float32),
            pltpu.VMEM((BLK_W + 16,), jnp.float32),
            pltpu.VMEM((16,), jnp.float32),
            pltpu.VMEM((16,), jnp.int32),
            pltpu.SemaphoreType.DMA,
            pltpu.SemaphoreType.DMA,
            pltpu.SemaphoreType.DMA,
        ],
    )(i, j, ebf, nbv)


def kernel(xyz, edge_index, edge_type_idx, edge_rest_lengths, edge_type_emb,
           W1, b1, W2, b2, non_edge_bias):
    f32 = jnp.float32
    i = edge_index[0].astype(jnp.int32)
    j = edge_index[1].astype(jnp.int32)
    tidx = edge_type_idx.astype(jnp.int32)
    rest = edge_rest_lengths.astype(f32)

    feat = _geom(xyz.astype(f32).reshape(-1), i, j, rest, tidx,
                 edge_type_emb.astype(f32).reshape(-1))

    w1t = jnp.zeros((32, NF), f32).at[:, :10].set(W1.astype(f32).T)
    eb = _mlp(w1t, b1.astype(f32).reshape(32, 1), W2.astype(f32).T.reshape(1, 32),
              b2.astype(f32).reshape(1), feat)

    nbv = jnp.broadcast_to(non_edge_bias.astype(f32).reshape(1), (16,))
    flat = _scatter(i, j, eb.reshape(BE), nbv)
    return flat.reshape(B, 1, N, N)


# speedup vs baseline: 25.7114x; 1.3097x over previous
import functools

import jax
import jax.numpy as jnp
from jax import lax
from jax.experimental import pallas as pl
from jax.experimental.pallas import tpu as pltpu
from jax.experimental.pallas import tpu_sc as plsc

B, N, E = 4, 2048, 32768
BE = B * E
NT = 32
EPT = E // NT
NF = 16

ROWS_PER_TILE = 256
TILES_PER_BATCH = N // ROWS_PER_TILE
BLK_ROWS = 16
NBLK = ROWS_PER_TILE // BLK_ROWS
BLK_W = BLK_ROWS * N
LIST_CAP = 10240
LIST_DUMP = LIST_CAP + 16

_MESH = dict(core_axis_name="c", subcore_axis_name="s", num_cores=2,
             num_subcores=16)


def _widx():
    return lax.axis_index("s") * 2 + lax.axis_index("c")


def _rsqrt_pos(x):
    ib = plsc.bitcast(x, jnp.int32)
    y = plsc.bitcast(jnp.int32(0x5F3759DF) - (ib >> 1), jnp.float32)
    for _ in range(3):
        y = y * (1.5 - 0.5 * x * y * y)
    return y


def _geom_body(xyz_hbm, i_hbm, j_hbm, rest_hbm, type_hbm, emb_hbm, feat_hbm,
               xyz_v, iv_v, jv_v, rest_v, type_v, emb_v, fb_v, sem):
    w = _widx()
    e0 = w * EPT
    cps_in = [
        pltpu.async_copy(xyz_hbm, xyz_v, sem),
        pltpu.async_copy(i_hbm.at[pl.ds(e0, EPT)], iv_v, sem),
        pltpu.async_copy(j_hbm.at[pl.ds(e0, EPT)], jv_v, sem),
        pltpu.async_copy(rest_hbm.at[pl.ds(e0, EPT)], rest_v, sem),
        pltpu.async_copy(type_hbm.at[pl.ds(e0, EPT)], type_v, sem),
        pltpu.async_copy(emb_hbm, emb_v, sem),
    ]
    for cp in cps_in:
        cp.wait()

    zero16 = jnp.zeros((16,), jnp.float32)

    def chunk(c, carry):
        sl = pl.ds(c * 16, 16)
        iv = iv_v[sl]
        jv = jv_v[sl]
        tv = type_v[sl] * 8
        restv = rest_v[sl]
        inv = 1.0 / (restv + 1e-9)
        iv3 = iv * 3
        jv3 = jv * 3
        for d in range(8):
            sd = plsc.load_gather(emb_v, [tv + d])
            for b in range(B):
                fb_v[b, 2 + d, sl] = sd
        for b in range(B):
            off = b * (N * 3)
            diffs = []
            for d in range(3):
                pi = plsc.load_gather(xyz_v, [iv3 + (off + d)])
                pj = plsc.load_gather(xyz_v, [jv3 + (off + d)])
                diffs.append(pi - pj)
            dist2 = diffs[0] * diffs[0] + diffs[1] * diffs[1] + diffs[2] * diffs[2]
            x = dist2 + 1e-9
            dist = x * _rsqrt_pos(x)
            delta = (dist - restv) * inv
            fb_v[b, 0, sl] = dist2
            fb_v[b, 1, sl] = delta
            for f in range(10, NF):
                fb_v[b, f, sl] = zero16
        return carry

    lax.fori_loop(0, EPT // 16, chunk, 0)

    cps = []
    for b in range(B):
        col0 = b * E + e0
        cps.append(pltpu.async_copy(fb_v.at[b], feat_hbm.at[:, pl.ds(col0, EPT)],
                                    sem))
    for cp in cps:
        cp.wait()


def _geom(xyz, i, j, rest, tidx, emb):
    mesh = plsc.VectorSubcoreMesh(**_MESH)
    return pl.kernel(
        _geom_body,
        out_type=jax.ShapeDtypeStruct((NF, BE), jnp.float32),
        mesh=mesh,
        compiler_params=pltpu.CompilerParams(needs_layout_passes=False),
        scratch_types=[
            pltpu.VMEM((B * N * 3,), jnp.float32),
            pltpu.VMEM((EPT,), jnp.int32),
            pltpu.VMEM((EPT,), jnp.int32),
            pltpu.VMEM((EPT,), jnp.float32),
            pltpu.VMEM((EPT,), jnp.int32),
            pltpu.VMEM((16 * 8,), jnp.float32),
            pltpu.VMEM((B, NF, EPT), jnp.float32),
            pltpu.SemaphoreType.DMA,
        ],
    )(xyz, i, j, rest, tidx, emb)


def _mlp_body(w1t_ref, b1_ref, w2t_ref, b2_ref, feat_ref, out_ref):
    f = feat_ref[...]
    mask = lax.broadcasted_iota(jnp.int32, (NF, 1), 0) < 10
    f = jnp.where(mask, f, 0.0)
    h = jnp.dot(w1t_ref[...], f, preferred_element_type=jnp.float32)
    h = h + b1_ref[...]
    h = h * (1.0 / (1.0 + jnp.exp(-h)))
    o = jnp.dot(w2t_ref[...], h, preferred_element_type=jnp.float32)
    out_ref[...] = o + b2_ref[0]


def _mlp(w1t, b1c, w2t, b2, feat):
    blk = 8192
    return pl.pallas_call(
        _mlp_body,
        grid=(BE // blk,),
        in_specs=[
            pl.BlockSpec((32, NF), lambda g: (0, 0)),
            pl.BlockSpec((32, 1), lambda g: (0, 0)),
            pl.BlockSpec((1, 32), lambda g: (0, 0)),
            pl.BlockSpec(memory_space=pltpu.SMEM),
            pl.BlockSpec((NF, blk), lambda g: (0, g)),
        ],
        out_specs=pl.BlockSpec((1, blk), lambda g: (0, g)),
        out_shape=jax.ShapeDtypeStruct((1, BE), jnp.float32),
    )(w1t, b1c, w2t, b2, feat)


def _scatter_body(i_hbm, j_hbm, eb_hbm, nb_hbm, out_hbm,
                  iv_st, jv_st, listI, listE, vals, bufA, bufB, nb_v, cnt_v,
                  semA, semB, sem_st):
    w = _widx()
    b = w // TILES_PER_BATCH
    r0 = (w % TILES_PER_BATCH) * ROWS_PER_TILE
    pltpu.sync_copy(nb_hbm, nb_v)
    nbvec = nb_v[...]
    iota = jnp.arange(16, dtype=jnp.int32)

    segs = [(p, s) for p in range(2) for s in range(E // 2048)]

    def _stage_in(t):
        slot = t % 2
        s = segs[t][1]
        return (pltpu.async_copy(i_hbm.at[pl.ds(s * 2048, 2048)],
                                 iv_st.at[pl.ds(slot * 2048, 2048)], sem_st),
                pltpu.async_copy(j_hbm.at[pl.ds(s * 2048, 2048)],
                                 jv_st.at[pl.ds(slot * 2048, 2048)], sem_st))

    pend = _stage_in(0)
    cnt = jnp.zeros((16,), jnp.int32)
    for t, (p, s) in enumerate(segs):
        for cp in pend:
            cp.wait()
        if t + 1 < len(segs):
            nxt = _stage_in(t + 1)
        slot = t % 2

        def sbody(c, cnt, p=p, s=s, slot=slot):
            iv = iv_st[pl.ds(slot * 2048 + c * 16, 16)]
            jv = jv_st[pl.ds(slot * 2048 + c * 16, 16)]
            rows = iv if p == 0 else jv
            cols = jv if p == 0 else iv
            m = (rows >= r0) & (rows < r0 + ROWS_PER_TILE)
            mi = m.astype(jnp.int32)
            cs = plsc.cumsum(mi)
            rank = cs - mi
            ok = m & (cnt < LIST_CAP)
            dest = jnp.where(ok, cnt + rank, LIST_DUMP)
            lidx = (rows - r0) * N + cols
            eid = (s * 2048 + c * 16) + iota
            plsc.store_scatter(listI, [dest], lidx)
            plsc.store_scatter(listE, [dest], eid)
            pc = plsc.all_reduce_population_count(m)
            return jnp.minimum(cnt + pc, LIST_CAP)

        cnt = lax.fori_loop(0, 128, sbody, cnt, unroll=2)
        if t + 1 < len(segs):
            pend = nxt

    for kk in range(4):
        plsc.store_scatter(listI, [cnt + iota + kk * 16],
                           jnp.full((16,), NBLK * BLK_W, jnp.int32))
        plsc.store_scatter(listE, [cnt + iota + kk * 16],
                           jnp.zeros((16,), jnp.int32))
    cnt_v[...] = cnt
    cnt_s = jnp.max(cnt_v[...])
    nch4 = (cnt_s + 63) // 64

    pltpu.sync_copy(eb_hbm.at[pl.ds(b * E, E)], bufB.at[pl.ds(0, E)])

    def vbody(n, carry):
        for u in range(4):
            base = n * 64 + u * 16
            eids = listE[pl.ds(base, 16)]
            v = plsc.load_gather(bufB, [eids])
            vals[pl.ds(base, 16)] = v
        return carry

    lax.fori_loop(0, nch4, vbody, 0)

    def _fill(buf):
        def fbody(tt, carry):
            buf[pl.ds(tt * 16, 16)] = nbvec
            return carry
        lax.fori_loop(0, BLK_W // 16, fbody, 0, unroll=8)

    _fill(bufA)
    _fill(bufB)

    cps = [None, None]
    for k in range(NBLK):
        buf = bufA if k % 2 == 0 else bufB
        sem = semA if k % 2 == 0 else semB
        if k >= 2:
            cps[k % 2].wait()
            prev = k - 2

            def rbody(n, carry, buf=buf, prev=prev):
                for u in range(4):
                    base = n * 64 + u * 16
                    li = listI[pl.ds(base, 16)]
                    m = (li >> 15) == prev
                    dest = jnp.where(m, li & (BLK_W - 1), BLK_W)
                    plsc.store_scatter(buf, [dest], nbvec)
                return carry

            lax.fori_loop(0, nch4, rbody, 0)

        def abody(n, carry, buf=buf, k=k):
            for u in range(4):
                base = n * 64 + u * 16
                li = listI[pl.ds(base, 16)]
                vv = vals[pl.ds(base, 16)]
                m = (li >> 15) == k
                dest = jnp.where(m, li & (BLK_W - 1), BLK_W)
                plsc.store_scatter(buf, [dest], vv)
            return carry

        lax.fori_loop(0, nch4, abody, 0)
        obase = (b * N + r0 + k * BLK_ROWS) * N
        cps[k % 2] = pltpu.async_copy(buf.at[pl.ds(0, BLK_W)],
                                      out_hbm.at[pl.ds(obase, BLK_W)], sem)
    cps[0].wait()
    cps[1].wait()


def _scatter(i, j, ebf, nbv):
    mesh = plsc.VectorSubcoreMesh(**_MESH)
    return pl.kernel(
        _scatter_body,
        out_type=jax.ShapeDtypeStruct((B * N * N,), jnp.float32),
        mesh=mesh,
        compiler_params=pltpu.CompilerParams(needs_layout_passes=False),
        scratch_types=[
            pltpu.VMEM((2 * 2048,), jnp.int32),
            pltpu.VMEM((2 * 2048,), jnp.int32),
            pltpu.VMEM((LIST_CAP + 80,), jnp.int32),
            pltpu.VMEM((LIST_CAP + 80,), jnp.int32),
            pltpu.VMEM((LIST_CAP + 80,), jnp.float32),
            pltpu.VMEM((BLK_W + 16,), jnp.float32),
            pltpu.VMEM((BLK_W + 16,), jnp.float32),
            pltpu.VMEM((16,), jnp.float32),
            pltpu.VMEM((16,), jnp.int32),
            pltpu.SemaphoreType.DMA,
            pltpu.SemaphoreType.DMA,
            pltpu.SemaphoreType.DMA,
        ],
    )(i, j, ebf, nbv)


def kernel(xyz, edge_index, edge_type_idx, edge_rest_lengths, edge_type_emb,
           W1, b1, W2, b2, non_edge_bias):
    f32 = jnp.float32
    i = edge_index[0].astype(jnp.int32)
    j = edge_index[1].astype(jnp.int32)
    tidx = edge_type_idx.astype(jnp.int32)
    rest = edge_rest_lengths.astype(f32)

    feat = _geom(xyz.astype(f32).reshape(-1), i, j, rest, tidx,
                 edge_type_emb.astype(f32).reshape(-1))

    w1t = jnp.zeros((32, NF), f32).at[:, :10].set(W1.astype(f32).T)
    eb = _mlp(w1t, b1.astype(f32).reshape(32, 1), W2.astype(f32).T.reshape(1, 32),
              b2.astype(f32).reshape(1), feat)

    nbv = jnp.broadcast_to(non_edge_bias.astype(f32).reshape(1), (16,))
    flat = _scatter(i, j, eb.reshape(BE), nbv)
    return flat.reshape(B, 1, N, N)
